# baseline (speedup 1.0000x reference)
import functools

import jax
import jax.numpy as jnp
from jax import lax
from jax.experimental import pallas as pl
from jax.experimental.pallas import tpu as pltpu
from jax.experimental.pallas import tpu_sc as plsc

D = 128
S = 1000
ES = 3200
NW = 32
EPS = 1e-6
C_EPS2 = float(D) * EPS * EPS
DUP_DIST = 1.1313708498984762e-05
E2 = 7.389056205749512

SW = 32
EW = 104


def _sc_gather_body(z_hbm, sidx_hbm, ei_hbm, ej_hbm, zs_out, zi_out, zj_out,
                    idx_v, rows_v, ei_v, ej_v, zi_v, zj_v,
                    sem_s, sem_i, sem_j):
    wid = lax.axis_index("s") * 2 + lax.axis_index("c")
    sbase = jnp.minimum(wid * SW, S - SW)
    ebase = jnp.minimum(wid * EW, ES - EW)

    ld_s = pltpu.async_copy(sidx_hbm.at[pl.ds(sbase, SW)], idx_v, sem_s)
    ld_i = pltpu.async_copy(ei_hbm.at[pl.ds(ebase, EW)], ei_v, sem_i)
    ld_j = pltpu.async_copy(ej_hbm.at[pl.ds(ebase, EW)], ej_v, sem_j)
    ld_s.wait()
    g_s = pltpu.async_copy(z_hbm.at[idx_v], rows_v, sem_s)
    ld_i.wait()
    g_i = pltpu.async_copy(z_hbm.at[ei_v], zi_v, sem_i)
    ld_j.wait()
    g_j = pltpu.async_copy(z_hbm.at[ej_v], zj_v, sem_j)
    g_s.wait()
    w_s = pltpu.async_copy(rows_v, zs_out.at[pl.ds(sbase, SW)], sem_s)
    g_i.wait()
    w_i = pltpu.async_copy(zi_v, zi_out.at[pl.ds(ebase, EW)], sem_i)
    g_j.wait()
    w_j = pltpu.async_copy(zj_v, zj_out.at[pl.ds(ebase, EW)], sem_j)
    w_s.wait()
    w_i.wait()
    w_j.wait()


@functools.cache
def _sc_gather_kernel():
    mesh = plsc.VectorSubcoreMesh(core_axis_name="c", subcore_axis_name="s")
    return pl.kernel(
        _sc_gather_body,
        out_type=(
            jax.ShapeDtypeStruct((S, D), jnp.float32),
            jax.ShapeDtypeStruct((ES, D), jnp.float32),
            jax.ShapeDtypeStruct((ES, D), jnp.float32),
        ),
        mesh=mesh,
        scratch_types=[
            pltpu.VMEM((SW,), jnp.int32),
            pltpu.VMEM((SW, D), jnp.float32),
            pltpu.VMEM((EW,), jnp.int32),
            pltpu.VMEM((EW,), jnp.int32),
            pltpu.VMEM((EW, D), jnp.float32),
            pltpu.VMEM((EW, D), jnp.float32),
            pltpu.SemaphoreType.DMA,
            pltpu.SemaphoreType.DMA,
            pltpu.SemaphoreType.DMA,
        ],
    )


def _tc_body(beta_ref, zs_ref, idxc_ref, idxr_ref, zi_hbm, zj_hbm, out_ref,
             zi_v, zj_v, sem_i, sem_j):
    cp_i = pltpu.make_async_copy(zi_hbm, zi_v, sem_i)
    cp_j = pltpu.make_async_copy(zj_hbm, zj_v, sem_j)
    cp_i.start()
    cp_j.start()
    beta = beta_ref[0, 0]
    zs = zs_ref[...]
    n = jnp.sum(zs * zs, axis=1, keepdims=True)
    s = jnp.sum(zs, axis=1, keepdims=True)
    a_col = n + (2.0 * EPS) * s + C_EPS2
    b_col = n - (2.0 * EPS) * s
    ones_col = jnp.ones((S, 1), jnp.float32)
    lhs = jnp.concatenate([zs, ones_col], axis=1)
    rhs = jnp.concatenate([-2.0 * zs, b_col], axis=1)
    g2 = lax.dot_general(lhs, rhs, (((1,), (1,)), ((), ())),
                         preferred_element_type=jnp.float32,
                         precision=lax.Precision.DEFAULT)
    dist = jnp.sqrt(jnp.maximum(a_col + g2, C_EPS2))
    eq = idxc_ref[...] == idxr_ref[...]
    dist = jnp.where(eq, DUP_DIST, dist)
    total = jnp.sum(jnp.exp(beta - dist)) - float(S) * jnp.exp(beta - DUP_DIST)
    z_pdist1 = 0.5 * E2 * total
    cp_i.wait()
    cp_j.wait()
    de = zi_v[...] - zj_v[...] + EPS
    e_d2 = jnp.sum(de * de, axis=1, keepdims=True)
    z_pdist2 = float(ES) * beta - jnp.sum(jnp.sqrt(e_d2))
    out_ref[0, 0] = z_pdist2 - z_pdist1


_tc_call = pl.pallas_call(
    _tc_body,
    out_shape=jax.ShapeDtypeStruct((1, 1), jnp.float32),
    in_specs=[
        pl.BlockSpec(memory_space=pltpu.SMEM),
        pl.BlockSpec(memory_space=pltpu.VMEM),
        pl.BlockSpec(memory_space=pltpu.VMEM),
        pl.BlockSpec(memory_space=pltpu.VMEM),
        pl.BlockSpec(memory_space=pltpu.MemorySpace.HBM),
        pl.BlockSpec(memory_space=pltpu.MemorySpace.HBM),
    ],
    out_specs=pl.BlockSpec(memory_space=pltpu.SMEM),
    scratch_shapes=[
        pltpu.VMEM((ES, D), jnp.float32),
        pltpu.VMEM((ES, D), jnp.float32),
        pltpu.SemaphoreType.DMA,
        pltpu.SemaphoreType.DMA,
    ],
)


def kernel(latent_Z, beta, sample_idx, sparse_sample_i, sparse_sample_j):
    zs, zi, zj = _sc_gather_kernel()(latent_Z, sample_idx,
                                     sparse_sample_i, sparse_sample_j)
    return _tc_call(
        beta.reshape(1, 1),
        zs,
        sample_idx.reshape(S, 1),
        sample_idx.reshape(1, S),
        zi,
        zj,
    )
